# baseline (speedup 1.0000x reference)
import functools

import jax
import jax.numpy as jnp
from jax import lax
from jax.experimental import pallas as pl
from jax.experimental.pallas import tpu as pltpu
from jax.experimental.pallas import tpu_sc as plsc

_N = 10000
_E = 320000
_D = 128
_NC = 2
_NT = 16
_NW = _NC * _NT
_C = 128
_H = _C // 2
_IG = 8
_NGR = 10
_NCH = _IG * _NGR
_EPT = _NCH * _C
_EPAD = _NW * _EPT
_TR = 10016
_AR = 10240
_ZR = 16
_APT = _AR // _NT
_HR = 10240


def _sc_aggregate(tbl, src4, dst4):
    mesh = plsc.VectorSubcoreMesh(core_axis_name="c", subcore_axis_name="s",
                                  num_cores=_NC, num_subcores=_NT)

    @functools.partial(
        pl.kernel,
        out_type=(
            jax.ShapeDtypeStruct((_NC, _AR, _D), jnp.float32),
            jax.ShapeDtypeStruct((_NW, _HR), jnp.float32),
        ),
        mesh=mesh,
        compiler_params=pltpu.CompilerParams(needs_layout_passes=False),
        scratch_types=(
            pltpu.VMEM_SHARED((_AR, _D), jnp.float32),
            pltpu.VMEM((_IG, _C), jnp.int32),
            pltpu.VMEM((_IG, _C), jnp.int32),
            pltpu.VMEM((_C, _D), jnp.float32),
            pltpu.VMEM((_C, _D), jnp.float32),
            pltpu.VMEM((_ZR, _D), jnp.float32),
            pltpu.VMEM((_HR,), jnp.float32),
            pltpu.SemaphoreType.DMA,
            pltpu.SemaphoreType.DMA,
            pltpu.SemaphoreType.DMA,
            pltpu.SemaphoreType.DMA,
        ),
    )
    def agg(tbl_h, src_h, dst_h, s_out, deg_out,
            s_acc, src_v, dst_v, gbuf0, gbuf1, zrows, hist,
            sg0, sg1, ss0, ss1):
        cid = lax.axis_index("c")
        sid = lax.axis_index("s")
        wid = cid * _NT + sid

        zv = jnp.zeros((16,), jnp.float32)

        def _zrow128(i, carry):
            for j in range(_D // 16):
                zrows[i, pl.ds(j * 16, 16)] = zv
            return carry

        lax.fori_loop(0, _ZR, _zrow128, 0)

        def _zhist(i, carry):
            hist[pl.ds(i * 16, 16)] = zv
            return carry

        lax.fori_loop(0, _HR // 16, _zhist, 0)

        abase = sid * _APT
        sems4 = (sg0, sg1, ss0, ss1)

        zd = [None, None, None, None]
        for k in range(_APT // _ZR):
            b = k & 3
            if k >= 4:
                zd[b].wait()
            zd[b] = pltpu.async_copy(
                zrows, s_acc.at[pl.ds(abase + k * _ZR, _ZR)], sems4[b])
        for b in range(4):
            zd[b].wait()
        plsc.subcore_barrier()

        ones16 = jnp.ones((16,), jnp.float32)
        bufs = (gbuf0, gbuf1)
        gsems = (sg0, sg1)
        ssems = (ss0, ss1)

        def _hist_count(j):
            def _deg(i, carry3):
                idx = dst_v[j, pl.ds(i * 16, 16)]
                plsc.addupdate_scatter(hist, [idx], ones16)
                return carry3

            lax.fori_loop(0, _C // 16, _deg, 0)

        def _group(g, carry):
            pltpu.sync_copy(src_h.at[cid, sid, pl.ds(g * _IG, _IG)], src_v)
            pltpu.sync_copy(dst_h.at[cid, sid, pl.ds(g * _IG, _IG)], dst_v)

            gd = [None, None]
            ge = [None, None]
            sd = [None, None]
            for step in range(_IG + 1):
                b = step & 1
                if step >= 2:
                    sd[b].wait()
                if step < _IG:
                    gd[b] = pltpu.async_copy(
                        tbl_h.at[src_v.at[step, pl.ds(0, _H)]],
                        bufs[b].at[pl.ds(0, _H)], gsems[b])
                    ge[b] = pltpu.async_copy(
                        tbl_h.at[src_v.at[step, pl.ds(_H, _H)]],
                        bufs[b].at[pl.ds(_H, _H)], gsems[b])
                if step >= 1:
                    j = step - 1
                    o = b ^ 1
                    gd[o].wait()
                    ge[o].wait()
                    sd[o] = pltpu.async_copy(
                        bufs[o], s_acc.at[dst_v.at[j]], ssems[o], add=True)
                    _hist_count(j)
            sd[(_IG - 1) & 1].wait()
            return carry

        lax.fori_loop(0, _NGR, _group, 0)
        plsc.subcore_barrier()

        pltpu.sync_copy(s_acc.at[pl.ds(abase, _APT)],
                        s_out.at[cid, pl.ds(abase, _APT)])
        pltpu.sync_copy(hist, deg_out.at[wid])

    return agg(tbl, src4, dst4)


def _tc_self(feat, w_self, bias):
    bn = 2000
    grid = (_N // bn,)

    def body(feat_ref, ws_ref, b_ref, o_ref):
        o_ref[...] = jnp.dot(feat_ref[...], ws_ref[...],
                             preferred_element_type=jnp.float32) + b_ref[...]

    return pl.pallas_call(
        body,
        grid=grid,
        in_specs=[
            pl.BlockSpec((bn, _D), lambda i: (i, 0)),
            pl.BlockSpec((_D, _D), lambda i: (0, 0)),
            pl.BlockSpec((1, _D), lambda i: (0, 0)),
        ],
        out_specs=pl.BlockSpec((bn, _D), lambda i: (i, 0)),
        out_shape=jax.ShapeDtypeStruct((_N, _D), jnp.float32),
    )(feat, w_self, bias)


def _tc_neigh(y0, s, deg2, w_neigh):
    bn = 2000
    grid = (_N // bn,)

    def body(y0_ref, s_ref, d_ref, wn_ref, o_ref):
        deg = jnp.sum(d_ref[...], axis=1, keepdims=True)
        r = 1.0 / jnp.maximum(deg, 1.0)
        h = (s_ref[0] + s_ref[1]) * r
        o_ref[...] = y0_ref[...] + jnp.dot(
            h, wn_ref[...], preferred_element_type=jnp.float32)

    return pl.pallas_call(
        body,
        grid=grid,
        in_specs=[
            pl.BlockSpec((bn, _D), lambda i: (i, 0)),
            pl.BlockSpec((_NC, bn, _D), lambda i: (0, i, 0)),
            pl.BlockSpec((bn, _NW), lambda i: (i, 0)),
            pl.BlockSpec((_D, _D), lambda i: (0, 0)),
        ],
        out_specs=pl.BlockSpec((bn, _D), lambda i: (i, 0)),
        out_shape=jax.ShapeDtypeStruct((_N, _D), jnp.float32),
    )(y0, s, deg2, w_neigh)


def kernel(feat, edge_index, W_self, b_self, W_neigh, b_neigh):
    src = edge_index[0].astype(jnp.int32)
    dst = edge_index[1].astype(jnp.int32)

    tbl = jnp.zeros((_TR, _D), jnp.float32).at[:_N, :].set(feat)

    pad = _EPAD - _E
    spill = _N + jnp.arange(pad, dtype=jnp.int32) % (_AR - _N)
    srcp = jnp.concatenate([src, jnp.full((pad,), _N, jnp.int32)])
    dstp = jnp.concatenate([dst, spill])
    src4 = srcp.reshape(_NC, _NT, _NCH, _C)
    dst4 = dstp.reshape(_NC, _NT, _NCH, _C)

    s, deg2 = _sc_aggregate(tbl, src4, dst4)
    bias = (b_self + b_neigh).reshape(1, _D)
    y0 = _tc_self(feat, W_self, bias)
    return _tc_neigh(y0, s, deg2.T, W_neigh)


# speedup vs baseline: 1.0278x; 1.0278x over previous
import functools

import jax
import jax.numpy as jnp
from jax import lax
from jax.experimental import pallas as pl
from jax.experimental.pallas import tpu as pltpu
from jax.experimental.pallas import tpu_sc as plsc

_N = 10000
_E = 320000
_D = 128
_NC = 2
_NT = 16
_NW = _NC * _NT
_C = 128
_H = _C // 2
_IG = 16
_NGR = 5
_NCH = _IG * _NGR
_EPT = _NCH * _C
_EPAD = _NW * _EPT
_TR = 10016
_AR = 10240
_ZR = 16
_APT = _AR // _NT
_HR = 10224
_SP = 224


def _sc_aggregate(tbl, src4, dst4):
    mesh = plsc.VectorSubcoreMesh(core_axis_name="c", subcore_axis_name="s",
                                  num_cores=_NC, num_subcores=_NT)

    @functools.partial(
        pl.kernel,
        out_type=(
            jax.ShapeDtypeStruct((_NC, _AR, _D), jnp.float32),
            jax.ShapeDtypeStruct((_NW, _HR), jnp.float32),
        ),
        mesh=mesh,
        compiler_params=pltpu.CompilerParams(needs_layout_passes=False),
        scratch_types=(
            pltpu.VMEM_SHARED((_AR, _D), jnp.float32),
            pltpu.VMEM((_IG, _C), jnp.int32),
            pltpu.VMEM((_IG, _C), jnp.int32),
            pltpu.VMEM((_C, _D), jnp.float32),
            pltpu.VMEM((_C, _D), jnp.float32),
            pltpu.VMEM((_ZR, _D), jnp.float32),
            pltpu.VMEM((_HR,), jnp.float32),
            pltpu.SemaphoreType.DMA,
            pltpu.SemaphoreType.DMA,
            pltpu.SemaphoreType.DMA,
            pltpu.SemaphoreType.DMA,
        ),
    )
    def agg(tbl_h, src_h, dst_h, s_out, deg_out,
            s_acc, src_v, dst_v, gbuf0, gbuf1, zrows, hist,
            sg0, sg1, ss0, ss1):
        cid = lax.axis_index("c")
        sid = lax.axis_index("s")
        wid = cid * _NT + sid

        zv = jnp.zeros((16,), jnp.float32)

        def _zrow128(i, carry):
            for j in range(_D // 16):
                zrows[i, pl.ds(j * 16, 16)] = zv
            return carry

        lax.fori_loop(0, _ZR, _zrow128, 0)

        def _zhist(i, carry):
            hist[pl.ds(i * 16, 16)] = zv
            return carry

        lax.fori_loop(0, _HR // 16, _zhist, 0)

        abase = sid * _APT
        sems4 = (sg0, sg1, ss0, ss1)

        zd = [None, None, None, None]
        for k in range(_APT // _ZR):
            b = k & 3
            if k >= 4:
                zd[b].wait()
            zd[b] = pltpu.async_copy(
                zrows, s_acc.at[pl.ds(abase + k * _ZR, _ZR)], sems4[b])
        for b in range(4):
            zd[b].wait()
        plsc.subcore_barrier()

        ones16 = jnp.ones((16,), jnp.float32)
        bufs = (gbuf0, gbuf1)
        gsems = (sg0, sg1)
        ssems = (ss0, ss1)

        def _hist_count(j):
            def _deg(i, carry3):
                idx = dst_v[j, pl.ds(i * 16, 16)]
                plsc.addupdate_scatter(hist, [idx], ones16)
                return carry3

            lax.fori_loop(0, _C // 16, _deg, 0)

        def _group(g, carry):
            pltpu.sync_copy(src_h.at[cid, sid, pl.ds(g * _IG, _IG)], src_v)
            pltpu.sync_copy(dst_h.at[cid, sid, pl.ds(g * _IG, _IG)], dst_v)

            gd = [None, None]
            ge = [None, None]
            sd = [None, None]
            for step in range(_IG + 1):
                b = step & 1
                if step >= 2:
                    sd[b].wait()
                if step < _IG:
                    gd[b] = pltpu.async_copy(
                        tbl_h.at[src_v.at[step, pl.ds(0, _H)]],
                        bufs[b].at[pl.ds(0, _H)], gsems[b])
                    ge[b] = pltpu.async_copy(
                        tbl_h.at[src_v.at[step, pl.ds(_H, _H)]],
                        bufs[b].at[pl.ds(_H, _H)], gsems[b])
                if step >= 1:
                    j = step - 1
                    o = b ^ 1
                    gd[o].wait()
                    ge[o].wait()
                    sd[o] = pltpu.async_copy(
                        bufs[o], s_acc.at[dst_v.at[j]], ssems[o], add=True)
                    _hist_count(j)
            sd[(_IG - 1) & 1].wait()
            return carry

        lax.fori_loop(0, _NGR, _group, 0)
        plsc.subcore_barrier()

        pltpu.sync_copy(s_acc.at[pl.ds(abase, _APT)],
                        s_out.at[cid, pl.ds(abase, _APT)])
        pltpu.sync_copy(hist, deg_out.at[wid])

    return agg(tbl, src4, dst4)


def _tc_combine(feat, s, deg2, w_self, w_neigh, bias):
    bn = 2000
    grid = (_N // bn,)

    def body(feat_ref, s_ref, d_ref, ws_ref, wn_ref, b_ref, o_ref):
        deg = jnp.sum(d_ref[...], axis=1, keepdims=True)
        r = 1.0 / jnp.maximum(deg, 1.0)
        h = (s_ref[0] + s_ref[1]) * r
        acc = jnp.dot(feat_ref[...], ws_ref[...],
                      preferred_element_type=jnp.float32)
        acc += jnp.dot(h, wn_ref[...], preferred_element_type=jnp.float32)
        o_ref[...] = acc + b_ref[...]

    return pl.pallas_call(
        body,
        grid=grid,
        in_specs=[
            pl.BlockSpec((bn, _D), lambda i: (i, 0)),
            pl.BlockSpec((_NC, bn, _D), lambda i: (0, i, 0)),
            pl.BlockSpec((bn, _NW), lambda i: (i, 0)),
            pl.BlockSpec((_D, _D), lambda i: (0, 0)),
            pl.BlockSpec((_D, _D), lambda i: (0, 0)),
            pl.BlockSpec((1, _D), lambda i: (0, 0)),
        ],
        out_specs=pl.BlockSpec((bn, _D), lambda i: (i, 0)),
        out_shape=jax.ShapeDtypeStruct((_N, _D), jnp.float32),
    )(feat, s, deg2, w_self, w_neigh, bias)


def kernel(feat, edge_index, W_self, b_self, W_neigh, b_neigh):
    src = edge_index[0].astype(jnp.int32)
    dst = edge_index[1].astype(jnp.int32)

    tbl = jnp.zeros((_TR, _D), jnp.float32).at[:_N, :].set(feat)

    pad = _EPAD - _E
    spill = _N + jnp.arange(pad, dtype=jnp.int32) % _SP
    srcp = jnp.concatenate([src, jnp.full((pad,), _N, jnp.int32)])
    dstp = jnp.concatenate([dst, spill])
    src4 = srcp.reshape(_NC, _NT, _NCH, _C)
    dst4 = dstp.reshape(_NC, _NT, _NCH, _C)

    s, deg2 = _sc_aggregate(tbl, src4, dst4)
    bias = (b_self + b_neigh).reshape(1, _D)
    return _tc_combine(feat, s, deg2.T, W_self, W_neigh, bias)
